# Initial kernel scaffold; baseline (speedup 1.0000x reference)
import jax
import jax.numpy as jnp
from jax.experimental import pallas as pl


def kernel(pred, target):
    raise NotImplementedError("write your pallas kernel here")


# speedup vs baseline: 1.8302x; 1.8302x over previous
import functools

import jax
import jax.numpy as jnp
from jax import lax
from jax.experimental import pallas as pl
from jax.experimental.pallas import tpu as pltpu

GAMMA_EXP = 2
ROWS_PER_BLOCK = 512


def _focal_body(pred_ref, tgt_ref, out_ref, cnt_ref, wsum_ref, *, bz, nclass):
    i = pl.program_id(0)
    nblocks = pl.num_programs(0)

    @pl.when(i == 0)
    def _init():
        cnt_ref[...] = jnp.zeros_like(cnt_ref)
        wsum_ref[...] = jnp.zeros_like(wsum_ref)

    x = pred_ref[...]
    t = tgt_ref[...]
    r = x.shape[0]

    m = jnp.max(x, axis=1, keepdims=True)
    e = jnp.exp(x - m)
    s = jnp.sum(e, axis=1, keepdims=True)

    cols = lax.broadcasted_iota(jnp.int32, (r, nclass), 1)
    onehot = (cols == t).astype(jnp.float32)
    pred_t = jnp.sum(x * onehot, axis=1, keepdims=True)

    logpt = pred_t - m - jnp.log(s)
    ce = -logpt
    pt = jnp.exp(logpt)
    f = (1.0 - pt) ** GAMMA_EXP * ce

    cnt_ref[...] += jnp.sum(onehot, axis=0, keepdims=True)
    wsum_ref[...] += jnp.sum(f * onehot, axis=0, keepdims=True)

    @pl.when(i == nblocks - 1)
    def _final():
        alpha = 1.0 - cnt_ref[...] / bz
        total = jnp.sum(alpha * wsum_ref[...]) / bz
        out_ref[...] = jnp.full((1, 1), total, jnp.float32)


def kernel(pred, target):
    bz, nclass = pred.shape
    r = ROWS_PER_BLOCK
    nblocks = bz // r
    t2d = target.astype(jnp.int32).reshape(bz, 1)

    out = pl.pallas_call(
        functools.partial(_focal_body, bz=float(bz), nclass=nclass),
        grid=(nblocks,),
        in_specs=[
            pl.BlockSpec((r, nclass), lambda i: (i, 0)),
            pl.BlockSpec((r, 1), lambda i: (i, 0)),
        ],
        out_specs=pl.BlockSpec((1, 1), lambda i: (0, 0)),
        out_shape=jax.ShapeDtypeStruct((1, 1), jnp.float32),
        scratch_shapes=[
            pltpu.VMEM((1, nclass), jnp.float32),
            pltpu.VMEM((1, nclass), jnp.float32),
        ],
    )(pred, t2d)
    return out.reshape(())
